# baseline (speedup 1.0000x reference)
import dataclasses
import functools

import jax
import jax.numpy as jnp
from jax import lax
from jax.experimental import pallas as pl
from jax.experimental.pallas import tpu as pltpu
from jax.experimental.pallas import tpu_sc as plsc

N_NODES = 10000
D = 256
HALF = 128
N_EDGES = 160000

NC = 2
NS = 16
ROWS = 10240
E_PAD = 163840
K = 128
E_PER_TEC = E_PAD // NS
NCHUNK = E_PER_TEC // K
E_PER_TEC_H = E_PAD // (NC * NS)
ROWS_PER_TEC = ROWS // NS

_mesh = plsc.VectorSubcoreMesh(core_axis_name="c", subcore_axis_name="s")

_cp = pltpu.CompilerParams()
if "needs_layout_passes" in pltpu.CompilerParams.__dataclass_fields__:
    _cp = dataclasses.replace(_cp, needs_layout_passes=False)


@functools.partial(
    pl.kernel,
    out_type=jax.ShapeDtypeStruct((NC * NS, ROWS), jnp.float32),
    mesh=_mesh,
    compiler_params=_cp,
    scratch_types=[
        pltpu.VMEM((E_PER_TEC_H,), jnp.int32),
        pltpu.VMEM((ROWS,), jnp.float32),
    ],
)
def _hist_kernel(dst_hbm, out_hbm, idx_v, hist_v):
    c = lax.axis_index("c")
    s = lax.axis_index("s")
    wid = c * NS + s
    pltpu.sync_copy(dst_hbm.at[pl.ds(wid * E_PER_TEC_H, E_PER_TEC_H)], idx_v)

    @pl.loop(0, ROWS, step=16)
    def _zero(i):
        hist_v[pl.ds(i, 16)] = jnp.zeros((16,), jnp.float32)

    ones = jnp.ones((16,), jnp.float32)

    @pl.loop(0, E_PER_TEC_H, step=16)
    def _accum(i):
        idx = idx_v[pl.ds(i, 16)]
        plsc.addupdate_scatter(hist_v, [idx], ones)

    pltpu.sync_copy(hist_v, out_hbm.at[wid])


NBUF = 2
NHALF = 2
NCHUNK2 = NCHUNK // NHALF

@functools.partial(
    pl.kernel,
    out_type=jax.ShapeDtypeStruct((NC, ROWS, HALF), jnp.float32),
    mesh=_mesh,
    scratch_types=[
        pltpu.VMEM((NCHUNK2, K), jnp.int32),
        pltpu.VMEM((NCHUNK2, K), jnp.int32),
        pltpu.VMEM((NBUF * K, HALF), jnp.float32),
        pltpu.VMEM_SHARED((ROWS, HALF), jnp.float32),
    ] + [pltpu.SemaphoreType.DMA] * (2 * NBUF + 1),
)
def _scatter_kernel(src2_hbm, dst_hbm, g_hbm, out_hbm,
                    src_half, dst_half, rows_v, acc, *sems):
    c = lax.axis_index("c")
    s = lax.axis_index("s")
    gsems = sems[:NBUF]
    ssems = sems[NBUF:2 * NBUF]
    zsem = sems[2 * NBUF]

    @pl.loop(0, NBUF * K)
    def _zrow(r):
        @pl.loop(0, HALF, step=16)
        def _zcol(j):
            rows_v[r, pl.ds(j, 16)] = jnp.zeros((16,), jnp.float32)

    row0 = s * ROWS_PER_TEC
    ZCOPIES = ROWS_PER_TEC // (NBUF * K)
    for z in range(ZCOPIES):
        pltpu.async_copy(rows_v, acc.at[pl.ds(row0 + z * NBUF * K, NBUF * K)],
                         zsem)
    rem = ROWS_PER_TEC - ZCOPIES * NBUF * K
    if rem:
        pltpu.async_copy(rows_v.at[pl.ds(0, rem)],
                         acc.at[pl.ds(row0 + ZCOPIES * NBUF * K, rem)], zsem)
    for z in range(ZCOPIES):
        pltpu.make_async_copy(rows_v, acc.at[pl.ds(row0, NBUF * K)],
                              zsem).wait()
    if rem:
        pltpu.make_async_copy(rows_v.at[pl.ds(0, rem)],
                              acc.at[pl.ds(row0, rem)], zsem).wait()

    plsc.subcore_barrier()

    def _buf(b):
        return rows_v.at[pl.ds(b * K, K)]

    def _gather_start(chunk, b):
        pltpu.async_copy(g_hbm.at[src_half.at[chunk]], _buf(b), gsems[b])

    def _gather_wait(b):
        pltpu.make_async_copy(g_hbm.at[src_half.at[0]], _buf(b),
                              gsems[b]).wait()

    def _scatter_start(chunk, b):
        pltpu.async_copy(_buf(b), acc.at[dst_half.at[chunk]], ssems[b],
                         add=True)

    def _scatter_wait(b):
        pltpu.make_async_copy(_buf(b), acc.at[dst_half.at[0]],
                              ssems[b]).wait()

    @pl.loop(0, NHALF)
    def _half(hf):
        pltpu.sync_copy(src2_hbm.at[c, s, pl.ds(hf * NCHUNK2, NCHUNK2)],
                        src_half)
        pltpu.sync_copy(dst_hbm.at[s, pl.ds(hf * NCHUNK2, NCHUNK2)], dst_half)

        for b in range(NBUF):
            _gather_start(b, b)

        @pl.loop(0, NCHUNK2 - NBUF, step=NBUF)
        def _chunks(j):
            for b in range(NBUF):
                _gather_wait(b)
                _scatter_start(j + b, b)
            for b in range(NBUF):
                _scatter_wait(b)
                _gather_start(j + b + NBUF, b)

        for b in range(NBUF):
            _gather_wait(b)
            _scatter_start(NCHUNK2 - NBUF + b, b)
        for b in range(NBUF):
            _scatter_wait(b)

    plsc.subcore_barrier()
    pltpu.sync_copy(acc.at[pl.ds(row0, ROWS_PER_TEC)],
                    out_hbm.at[c, pl.ds(row0, ROWS_PER_TEC)])


def _a1_body(hist_ref, x_ref, w1_ref, gcat_ref, dinv_ref):
    deg = jnp.sum(hist_ref[...], axis=1, keepdims=True)[:N_NODES] + 1.0
    dinv = lax.rsqrt(deg)
    dinv_ref[...] = dinv
    h = jnp.dot(x_ref[...], w1_ref[...], preferred_element_type=jnp.float32)
    g = h * dinv
    gcat_ref[pl.ds(0, N_NODES), :] = g[:, :HALF]
    gcat_ref[pl.ds(ROWS, N_NODES), :] = g[:, HALF:]
    zpad = jnp.zeros((ROWS - N_NODES, HALF), jnp.float32)
    gcat_ref[pl.ds(N_NODES, ROWS - N_NODES), :] = zpad
    gcat_ref[pl.ds(ROWS + N_NODES, ROWS - N_NODES), :] = zpad


def _mid_body(s0_ref, s1_ref, gcat_ref, dinv_ref, b1_ref, w2_ref, gcat2_ref):
    dinv = dinv_ref[...]
    u0 = s0_ref[pl.ds(0, N_NODES), :] + gcat_ref[pl.ds(0, N_NODES), :]
    u1 = s1_ref[pl.ds(0, N_NODES), :] + gcat_ref[pl.ds(ROWS, N_NODES), :]
    u = jnp.concatenate([u0, u1], axis=1)
    z = jnp.maximum(u * dinv + b1_ref[...], 0.0)
    h2 = jnp.dot(z, w2_ref[...], preferred_element_type=jnp.float32)
    g2 = h2 * dinv
    gcat2_ref[pl.ds(0, N_NODES), :] = g2[:, :HALF]
    gcat2_ref[pl.ds(ROWS, N_NODES), :] = g2[:, HALF:]
    zpad = jnp.zeros((ROWS - N_NODES, HALF), jnp.float32)
    gcat2_ref[pl.ds(N_NODES, ROWS - N_NODES), :] = zpad
    gcat2_ref[pl.ds(ROWS + N_NODES, ROWS - N_NODES), :] = zpad


def _final_body(s0_ref, s1_ref, gcat_ref, dinv_ref, b2_ref, out_ref):
    u0 = s0_ref[pl.ds(0, N_NODES), :] + gcat_ref[pl.ds(0, N_NODES), :]
    u1 = s1_ref[pl.ds(0, N_NODES), :] + gcat_ref[pl.ds(ROWS, N_NODES), :]
    u = jnp.concatenate([u0, u1], axis=1)
    out_ref[...] = u * dinv_ref[...] + b2_ref[...]


def kernel(x, edge_index, W1, b1, W2, b2):
    src = edge_index[0].astype(jnp.int32)
    dst = edge_index[1].astype(jnp.int32)

    pad = E_PAD - N_EDGES
    src_p = jnp.concatenate([src, jnp.zeros((pad,), jnp.int32)])
    dst_p = jnp.concatenate([dst, jnp.full((pad,), N_NODES, jnp.int32)])
    src2 = jnp.stack([src_p, src_p + ROWS]).reshape(NC, NS, NCHUNK, K)
    dst_c = dst_p.reshape(NS, NCHUNK, K)

    hist = _hist_kernel(dst_p)
    hist_t = hist.T

    b1r = b1.reshape(1, D)
    b2r = b2.reshape(1, D)

    gcat1, dinv = pl.pallas_call(
        _a1_body,
        out_shape=(
            jax.ShapeDtypeStruct((2 * ROWS, HALF), jnp.float32),
            jax.ShapeDtypeStruct((N_NODES, 1), jnp.float32),
        ),
    )(hist_t, x, W1)

    s1 = _scatter_kernel(src2, dst_c, gcat1)

    gcat2 = pl.pallas_call(
        _mid_body,
        out_shape=jax.ShapeDtypeStruct((2 * ROWS, HALF), jnp.float32),
    )(s1[0], s1[1], gcat1, dinv, b1r, W2)

    s2 = _scatter_kernel(src2, dst_c, gcat2)

    out = pl.pallas_call(
        _final_body,
        out_shape=jax.ShapeDtypeStruct((N_NODES, D), jnp.float32),
    )(s2[0], s2[1], gcat2, dinv, b2r)

    return out


# speedup vs baseline: 1.1722x; 1.1722x over previous
import dataclasses
import functools

import jax
import jax.numpy as jnp
from jax import lax
from jax.experimental import pallas as pl
from jax.experimental.pallas import tpu as pltpu
from jax.experimental.pallas import tpu_sc as plsc

N_NODES = 10000
D = 256
HALF = 128
N_EDGES = 160000

NC = 2
NS = 16
ROWS = 10240
E_PAD = 163840
K = 128
E_PER_TEC = E_PAD // NS
NCHUNK = E_PER_TEC // K
E_PER_TEC_H = E_PAD // (NC * NS)
ROWS_PER_TEC = ROWS // NS

_mesh = plsc.VectorSubcoreMesh(core_axis_name="c", subcore_axis_name="s")

_cp = pltpu.CompilerParams()
if "needs_layout_passes" in pltpu.CompilerParams.__dataclass_fields__:
    _cp = dataclasses.replace(_cp, needs_layout_passes=False)


@functools.partial(
    pl.kernel,
    out_type=jax.ShapeDtypeStruct((NC * NS, ROWS), jnp.float32),
    mesh=_mesh,
    compiler_params=_cp,
    scratch_types=[
        pltpu.VMEM((E_PER_TEC_H,), jnp.int32),
        pltpu.VMEM((ROWS,), jnp.float32),
    ],
)
def _hist_kernel(dst_hbm, out_hbm, idx_v, hist_v):
    c = lax.axis_index("c")
    s = lax.axis_index("s")
    wid = c * NS + s
    pltpu.sync_copy(dst_hbm.at[pl.ds(wid * E_PER_TEC_H, E_PER_TEC_H)], idx_v)

    @pl.loop(0, ROWS, step=16)
    def _zero(i):
        hist_v[pl.ds(i, 16)] = jnp.zeros((16,), jnp.float32)

    ones = jnp.ones((16,), jnp.float32)

    @pl.loop(0, E_PER_TEC_H, step=16)
    def _accum(i):
        idx = idx_v[pl.ds(i, 16)]
        plsc.addupdate_scatter(hist_v, [idx], ones)

    pltpu.sync_copy(hist_v, out_hbm.at[wid])


NBUF = 2
NHALF = 2
NCHUNK2 = NCHUNK // NHALF

@functools.partial(
    pl.kernel,
    out_type=jax.ShapeDtypeStruct((NC, ROWS, HALF), jnp.float32),
    mesh=_mesh,
    scratch_types=[
        pltpu.VMEM((NCHUNK2, K), jnp.int32),
        pltpu.VMEM((NCHUNK2, K), jnp.int32),
        pltpu.VMEM((NBUF * K, HALF), jnp.float32),
        pltpu.VMEM_SHARED((ROWS, HALF), jnp.float32),
    ] + [pltpu.SemaphoreType.DMA] * (NBUF + 1),
)
def _scatter_kernel(src2_hbm, dst_hbm, g_hbm, out_hbm,
                    src_half, dst_half, rows_v, acc, *sems):
    c = lax.axis_index("c")
    s = lax.axis_index("s")
    gsems = sems[:NBUF]
    isem = sems[NBUF]

    row0 = s * ROWS_PER_TEC
    pltpu.async_copy(g_hbm.at[pl.ds(c * ROWS + row0, ROWS_PER_TEC)],
                     acc.at[pl.ds(row0, ROWS_PER_TEC)], isem)
    pltpu.make_async_copy(g_hbm.at[pl.ds(0, ROWS_PER_TEC)],
                          acc.at[pl.ds(row0, ROWS_PER_TEC)], isem).wait()

    plsc.subcore_barrier()

    def _buf(b):
        return rows_v.at[pl.ds(b * K, K)]

    def _gather_start(chunk, b):
        pltpu.async_copy(g_hbm.at[src_half.at[chunk]], _buf(b), gsems[b])

    def _gather_wait(b):
        pltpu.make_async_copy(g_hbm.at[src_half.at[0]], _buf(b),
                              gsems[b]).wait()

    def _scatter(chunk, b):
        pltpu.sync_copy(_buf(b), acc.at[dst_half.at[chunk]], add=True)

    @pl.loop(0, NHALF)
    def _half(hf):
        pltpu.sync_copy(src2_hbm.at[c, s, pl.ds(hf * NCHUNK2, NCHUNK2)],
                        src_half)
        pltpu.sync_copy(dst_hbm.at[s, pl.ds(hf * NCHUNK2, NCHUNK2)], dst_half)

        for b in range(NBUF):
            _gather_start(b, b)

        @pl.loop(0, NCHUNK2 - NBUF, step=NBUF)
        def _chunks(j):
            for b in range(NBUF):
                _gather_wait(b)
                _scatter(j + b, b)
                _gather_start(j + b + NBUF, b)

        for b in range(NBUF):
            _gather_wait(b)
            _scatter(NCHUNK2 - NBUF + b, b)

    plsc.subcore_barrier()
    pltpu.sync_copy(acc.at[pl.ds(row0, ROWS_PER_TEC)],
                    out_hbm.at[c, pl.ds(row0, ROWS_PER_TEC)])


def _mm1_body(x_ref, w1_ref, h_ref):
    h_ref[...] = jnp.dot(x_ref[...], w1_ref[...],
                         preferred_element_type=jnp.float32)


def _combine_body(hist_ref, h_ref, gcat_ref, dinv_ref):
    deg = jnp.sum(hist_ref[...], axis=1, keepdims=True)[:N_NODES] + 1.0
    dinv = lax.rsqrt(deg)
    dinv_ref[...] = dinv
    g = h_ref[...] * dinv
    gcat_ref[pl.ds(0, N_NODES), :] = g[:, :HALF]
    gcat_ref[pl.ds(ROWS, N_NODES), :] = g[:, HALF:]
    zpad = jnp.zeros((ROWS - N_NODES, HALF), jnp.float32)
    gcat_ref[pl.ds(N_NODES, ROWS - N_NODES), :] = zpad
    gcat_ref[pl.ds(ROWS + N_NODES, ROWS - N_NODES), :] = zpad


def _mid_body(s0_ref, s1_ref, dinv_ref, b1_ref, w2_ref, gcat2_ref):
    dinv = dinv_ref[...]
    u = jnp.concatenate([s0_ref[pl.ds(0, N_NODES), :],
                         s1_ref[pl.ds(0, N_NODES), :]], axis=1)
    z = jnp.maximum(u * dinv + b1_ref[...], 0.0)
    h2 = jnp.dot(z, w2_ref[...], preferred_element_type=jnp.float32)
    g2 = h2 * dinv
    gcat2_ref[pl.ds(0, N_NODES), :] = g2[:, :HALF]
    gcat2_ref[pl.ds(ROWS, N_NODES), :] = g2[:, HALF:]
    zpad = jnp.zeros((ROWS - N_NODES, HALF), jnp.float32)
    gcat2_ref[pl.ds(N_NODES, ROWS - N_NODES), :] = zpad
    gcat2_ref[pl.ds(ROWS + N_NODES, ROWS - N_NODES), :] = zpad


def _final_body(s0_ref, s1_ref, dinv_ref, b2_ref, out_ref):
    u = jnp.concatenate([s0_ref[pl.ds(0, N_NODES), :],
                         s1_ref[pl.ds(0, N_NODES), :]], axis=1)
    out_ref[...] = u * dinv_ref[...] + b2_ref[...]


def kernel(x, edge_index, W1, b1, W2, b2):
    src = edge_index[0].astype(jnp.int32)
    dst = edge_index[1].astype(jnp.int32)

    pad = E_PAD - N_EDGES
    src_p = jnp.concatenate([src, jnp.zeros((pad,), jnp.int32)])
    dst_p = jnp.concatenate([dst, jnp.full((pad,), N_NODES, jnp.int32)])
    src2 = jnp.stack([src_p, src_p + ROWS]).reshape(NC, NS, NCHUNK, K)
    dst_c = dst_p.reshape(NS, NCHUNK, K)

    hist = _hist_kernel(dst_p)
    hist_t = hist.T

    h = pl.pallas_call(
        _mm1_body,
        out_shape=jax.ShapeDtypeStruct((N_NODES, D), jnp.float32),
    )(x, W1)

    b1r = b1.reshape(1, D)
    b2r = b2.reshape(1, D)

    gcat1, dinv = pl.pallas_call(
        _combine_body,
        out_shape=(
            jax.ShapeDtypeStruct((2 * ROWS, HALF), jnp.float32),
            jax.ShapeDtypeStruct((N_NODES, 1), jnp.float32),
        ),
    )(hist_t, h)

    s1 = _scatter_kernel(src2, dst_c, gcat1)

    gcat2 = pl.pallas_call(
        _mid_body,
        out_shape=jax.ShapeDtypeStruct((2 * ROWS, HALF), jnp.float32),
    )(s1[0], s1[1], dinv, b1r, W2)

    s2 = _scatter_kernel(src2, dst_c, gcat2)

    out = pl.pallas_call(
        _final_body,
        out_shape=jax.ShapeDtypeStruct((N_NODES, D), jnp.float32),
    )(s2[0], s2[1], dinv, b2r)

    return out
